# Initial kernel scaffold; baseline (speedup 1.0000x reference)
import jax
import jax.numpy as jnp
from jax.experimental import pallas as pl


def kernel(z, edge_index, batch, z_emb, W0, b0, W1, b1, W2, b2, W3, b3, conv1_w, conv1_b, conv2_w, conv2_b, lin1_w, lin1_b, lin2_w, lin2_b):
    raise NotImplementedError("write your pallas kernel here")


# speedup vs baseline: 1.0994x; 1.0994x over previous
import functools

import jax
import jax.numpy as jnp
from jax.experimental import pallas as pl

_NUM_GRAPHS = 256
_K = 30


def _mm_kernel(x_ref, w_ref, b_ref, o_ref, *, act):
    y = jnp.dot(x_ref[...], w_ref[...], preferred_element_type=jnp.float32)
    y = y + b_ref[...]
    if act == "relu":
        y = jnp.maximum(y, 0.0)
    elif act == "tanh":
        y = jnp.tanh(y)
    o_ref[...] = y


def _mm(x, w, b, act=None):
    n, k = x.shape
    m = w.shape[1]
    mp = ((m + 127) // 128) * 128
    wp = jnp.pad(w, ((0, 0), (0, mp - m)))
    bp = jnp.pad(b.reshape(1, -1), ((0, 0), (0, mp - m)))
    blk = 4096 if n >= 4096 else ((n + 7) // 8) * 8
    np_ = ((n + blk - 1) // blk) * blk
    xp = jnp.pad(x, ((0, np_ - n), (0, 0)))
    out = pl.pallas_call(
        functools.partial(_mm_kernel, act=act),
        grid=(np_ // blk,),
        in_specs=[
            pl.BlockSpec((blk, k), lambda i: (i, 0)),
            pl.BlockSpec((k, mp), lambda i: (0, 0)),
            pl.BlockSpec((1, mp), lambda i: (0, 0)),
        ],
        out_specs=pl.BlockSpec((blk, mp), lambda i: (i, 0)),
        out_shape=jax.ShapeDtypeStruct((np_, mp), jnp.float32),
    )(xp, wp, bp)
    return out[:n, :m]


def _gcn_act_kernel(agg_ref, h_ref, d_ref, b_ref, o_ref):
    o_ref[...] = jnp.tanh(agg_ref[...] + d_ref[...] * h_ref[...] + b_ref[...])


def _gcn_act(agg, h, dinv2, b):
    n, m = agg.shape
    blk = 4096
    np_ = ((n + blk - 1) // blk) * blk
    pad = ((0, np_ - n), (0, 0))
    out = pl.pallas_call(
        _gcn_act_kernel,
        grid=(np_ // blk,),
        in_specs=[
            pl.BlockSpec((blk, m), lambda i: (i, 0)),
            pl.BlockSpec((blk, m), lambda i: (i, 0)),
            pl.BlockSpec((blk, 1), lambda i: (i, 0)),
            pl.BlockSpec((1, m), lambda i: (0, 0)),
        ],
        out_specs=pl.BlockSpec((blk, m), lambda i: (i, 0)),
        out_shape=jax.ShapeDtypeStruct((np_, m), jnp.float32),
    )(jnp.pad(agg, pad), jnp.pad(h, pad), jnp.pad(dinv2, pad),
      b.reshape(1, -1))
    return out[:n]


def kernel(z, edge_index, batch, z_emb, W0, b0, W1, b1, W2, b2, W3, b3,
           conv1_w, conv1_b, conv2_w, conv2_b, lin1_w, lin1_b,
           lin2_w, lin2_b):
    n = z.shape[0]
    src, dst = edge_index[0], edge_index[1]

    deg = jnp.bincount(dst, length=n).astype(jnp.float32) + 1.0
    dinv = jax.lax.rsqrt(deg)
    norm = dinv[src] * dinv[dst]
    dinv2 = (dinv * dinv)[:, None]

    table0 = _mm(z_emb, W0, jnp.zeros((W0.shape[1],), jnp.float32))
    h = table0[z]

    weights = [W1, W2, W3]
    biases = [b0, b1, b2, b3]
    outs = []
    for li in range(4):
        msg = h[src] * norm[:, None]
        agg = jnp.zeros((n, h.shape[1]), jnp.float32).at[dst].add(msg)
        out_i = _gcn_act(agg, h, dinv2, biases[li])
        outs.append(out_i)
        if li < 3:
            w_next = weights[li]
            h = _mm(out_i, w_next,
                    jnp.zeros((w_next.shape[1],), jnp.float32))

    x_cat = jnp.concatenate(outs, axis=1)
    score = outs[3][:, 0]

    counts = jnp.bincount(batch, length=_NUM_GRAPHS)
    starts = jnp.concatenate(
        [jnp.zeros((1,), counts.dtype), jnp.cumsum(counts)[:-1]])
    idx0 = jnp.arange(n, dtype=jnp.int32)
    _, _, perm = jax.lax.sort((batch, -score, idx0), num_keys=2,
                              is_stable=True)
    sel = starts[:, None] + jnp.arange(_K, dtype=counts.dtype)[None, :]
    node_idx = perm[jnp.minimum(sel, n - 1)]
    valid = jnp.arange(_K, dtype=counts.dtype)[None, :] < counts[:, None]
    dense = x_cat[node_idx]
    dense = jnp.where(valid[:, :, None], dense, 0.0)

    d_lat = x_cat.shape[1]
    t1 = _mm(dense.reshape(_NUM_GRAPHS * _K, d_lat),
             conv1_w[:, 0, :].T, conv1_b, act="relu")
    t1 = t1.reshape(_NUM_GRAPHS, _K, 16)
    pool = jnp.maximum(t1[:, 0::2, :], t1[:, 1::2, :])

    taps = jnp.stack([pool[:, t:t + 11, :] for t in range(5)], axis=2)
    col = taps.reshape(_NUM_GRAPHS * 11, 5 * 16)
    wcol = conv2_w.transpose(2, 1, 0).reshape(5 * 16, 32)
    t2 = _mm(col, wcol, conv2_b, act="relu")
    t2 = t2.reshape(_NUM_GRAPHS, 11, 32).transpose(0, 2, 1)
    t2 = t2.reshape(_NUM_GRAPHS, 11 * 32)

    t3 = _mm(t2, lin1_w, lin1_b, act="relu")
    return _mm(t3, lin2_w, lin2_b)
